# baseline (speedup 1.0000x reference)
import functools

import jax
import jax.numpy as jnp
from jax.experimental import pallas as pl
from jax.experimental.pallas import tpu as pltpu

_ALPHA = 0.2
_EPS = 1e-30
_NH = 8
_DAUG = 128


def _proj_kernel(x_ref, w_ref, a1_ref, a2_ref, wh_ref, e1_ref, e2t_ref,
                 *, nheads, dhead):
    wh = jnp.dot(x_ref[...], w_ref[...], preferred_element_type=jnp.float32)
    br = wh.shape[0]
    for h in range(nheads):
        sl = wh[:, h * dhead : (h + 1) * dhead].astype(jnp.bfloat16)
        wh_ref[:, h * _DAUG : h * _DAUG + dhead] = sl
        wh_ref[:, h * _DAUG + dhead : (h + 1) * _DAUG] = jnp.zeros(
            (br, _DAUG - dhead), jnp.bfloat16)
        wh_ref[:, h * _DAUG + dhead : h * _DAUG + dhead + 1] = jnp.ones(
            (br, 1), jnp.bfloat16)
    e1_ref[...] = jnp.dot(wh, a1_ref[...], preferred_element_type=jnp.float32)
    e2 = jnp.dot(wh, a2_ref[...], preferred_element_type=jnp.float32)
    e2t_ref[...] = e2.T


def _project(xin, wcat, a1, a2, br, nheads, dhead):
    npad, din = xin.shape
    dout = wcat.shape[1]
    daug = nheads * _DAUG
    return pl.pallas_call(
        functools.partial(_proj_kernel, nheads=nheads, dhead=dhead),
        grid=(npad // br,),
        in_specs=[
            pl.BlockSpec((br, din), lambda i: (i, 0)),
            pl.BlockSpec((din, dout), lambda i: (0, 0)),
            pl.BlockSpec((dout, _NH), lambda i: (0, 0)),
            pl.BlockSpec((dout, _NH), lambda i: (0, 0)),
        ],
        out_specs=[
            pl.BlockSpec((br, daug), lambda i: (i, 0)),
            pl.BlockSpec((br, _NH), lambda i: (i, 0)),
            pl.BlockSpec((_NH, br), lambda i: (0, i)),
        ],
        out_shape=[
            jax.ShapeDtypeStruct((npad, daug), jnp.bfloat16),
            jax.ShapeDtypeStruct((npad, _NH), jnp.float32),
            jax.ShapeDtypeStruct((_NH, npad), jnp.float32),
        ],
    )(xin, wcat, a1, a2)


def _att_kernel(adj_ref, wh_ref, rs_ref, cs_ref, out_ref,
                *, nheads, dhead, ncols, bc, br, nreal, concat):
    i = pl.program_id(0)
    j = pl.program_id(1)

    @pl.when(j == 0)
    def _init():
        out_ref[...] = jnp.zeros_like(out_ref)

    adj = adj_ref[...]
    adj = jax.lax.cond(
        j == ncols - 1,
        lambda a: jnp.where(
            jax.lax.broadcasted_iota(jnp.int32, a.shape, 1) + j * bc < nreal,
            a, 0.0),
        lambda a: a,
        adj,
    )
    adjb = adj if adj.dtype == jnp.bfloat16 else adj.astype(jnp.bfloat16)

    for h in range(nheads):
        lo = h * _DAUG
        av = rs_ref[:, h : h + 1]
        cv = rs_ref[:, _NH + h : _NH + h + 1]
        bv = cs_ref[h : h + 1, :]
        dv = cs_ref[_NH + h : _NH + h + 1, :]
        p = adjb * jnp.maximum(av * bv, cv * dv)
        out_ref[:, lo : lo + _DAUG] += jnp.dot(
            p, wh_ref[:, lo : lo + _DAUG],
            preferred_element_type=jnp.float32,
        )

    @pl.when(j == ncols - 1)
    def _finish():
        rmask = jax.lax.broadcasted_iota(jnp.int32, (br, 1), 0) + i * br < nreal
        for h in range(nheads):
            lo = h * _DAUG
            den = out_ref[:, lo + dhead : lo + dhead + 1]
            v = out_ref[:, lo : lo + dhead] / (den + _EPS)
            if concat:
                v = jnp.where(v > 0, v, jnp.exp(v) - 1.0)
            out_ref[:, lo : lo + dhead] = jnp.where(rmask, v, 0.0)
            out_ref[:, lo + dhead : lo + _DAUG] = jnp.zeros(
                (br, _DAUG - dhead), jnp.float32)


def _attention(adj, wh, rs, cs, nheads, dhead, concat, br, bc):
    nreal = adj.shape[0]
    npad = wh.shape[0]
    daug = wh.shape[1]
    nrows = npad // br
    ncols = npad // bc
    return pl.pallas_call(
        functools.partial(
            _att_kernel, nheads=nheads, dhead=dhead, ncols=ncols, bc=bc,
            br=br, nreal=nreal, concat=concat,
        ),
        grid=(nrows, ncols),
        in_specs=[
            pl.BlockSpec((br, bc), lambda i, j: (i, j)),
            pl.BlockSpec((bc, daug), lambda i, j: (j, 0)),
            pl.BlockSpec((br, 2 * _NH), lambda i, j: (i, 0)),
            pl.BlockSpec((2 * _NH, bc), lambda i, j: (0, j)),
        ],
        out_specs=pl.BlockSpec((br, daug), lambda i, j: (i, 0)),
        out_shape=jax.ShapeDtypeStruct((npad, daug), jnp.float32),
        compiler_params=pltpu.CompilerParams(
            dimension_semantics=("parallel", "arbitrary")
        ),
    )(adj, wh, rs, cs)


def _leaky(v):
    return jnp.maximum(v, _ALPHA * v)


def _factors(e1, e2t):
    m2 = jnp.max(e2t, axis=1)
    sh = _leaky(e1 + m2[None, :])
    rs = jnp.concatenate(
        [jnp.exp(e1 + m2[None, :] - sh), jnp.exp(_ALPHA * (e1 + m2[None, :]) - sh)],
        axis=1,
    ).astype(jnp.bfloat16)
    cs = jnp.concatenate(
        [jnp.exp(e2t - m2[:, None]), jnp.exp(_ALPHA * (e2t - m2[:, None]))],
        axis=0,
    ).astype(jnp.bfloat16)
    return rs, cs


def kernel(x, adj, W_heads, a_heads, W_out, a_out):
    n, nfeat = x.shape
    nheads, _, nhid = W_heads.shape
    nclass = W_out.shape[1]

    br, bc = 2048, 1024
    npad = -(-n // max(br, bc)) * max(br, bc)

    xp = jnp.pad(x, ((0, npad - n), (0, 0)))

    wcat = jnp.transpose(W_heads, (1, 0, 2)).reshape(nfeat, nheads * nhid)
    a1 = jnp.zeros((nheads * nhid, _NH), jnp.float32)
    a2 = jnp.zeros((nheads * nhid, _NH), jnp.float32)
    for h in range(nheads):
        a1 = a1.at[h * nhid : (h + 1) * nhid, h].set(a_heads[h, :nhid, 0])
        a2 = a2.at[h * nhid : (h + 1) * nhid, h].set(a_heads[h, nhid:, 0])
    wh1, e11, e2t1 = _project(xp, wcat, a1, a2, br, nheads, nhid)
    rs1, cs1 = _factors(e11, e2t1)
    h1 = _attention(adj, wh1, rs1, cs1, nheads, nhid, True, br, bc)

    wcat2 = jnp.zeros((nheads * _DAUG, nclass), jnp.float32)
    for h in range(nheads):
        wcat2 = wcat2.at[h * _DAUG : h * _DAUG + nhid, :].set(
            W_out[h * nhid : (h + 1) * nhid, :])
    b1 = jnp.zeros((nclass, _NH), jnp.float32).at[:, 0].set(a_out[:nclass, 0])
    b2 = jnp.zeros((nclass, _NH), jnp.float32).at[:, 0].set(a_out[nclass:, 0])
    wh2, e12, e2t2 = _project(h1, wcat2, b1, b2, br, 1, nclass)
    rs2, cs2 = _factors(e12, e2t2)
    z = _attention(adj, wh2, rs2, cs2, 1, nclass, True, br, bc)
    return z[:n, :nclass][None, :, :]


# speedup vs baseline: 1.0140x; 1.0140x over previous
import functools

import jax
import jax.numpy as jnp
from jax.experimental import pallas as pl
from jax.experimental.pallas import tpu as pltpu

_ALPHA = 0.2
_EPS = 1e-30
_NH = 8
_DAUG = 128


def _proj_kernel(x_ref, w_ref, a1_ref, a2_ref, wh_ref, e1_ref, e2t_ref,
                 *, nheads, dhead):
    wh = jnp.dot(x_ref[...], w_ref[...], preferred_element_type=jnp.float32)
    br = wh.shape[0]
    for h in range(nheads):
        sl = wh[:, h * dhead : (h + 1) * dhead].astype(jnp.bfloat16)
        wh_ref[:, h * _DAUG : h * _DAUG + dhead] = sl
        wh_ref[:, h * _DAUG + dhead : (h + 1) * _DAUG] = jnp.zeros(
            (br, _DAUG - dhead), jnp.bfloat16)
        wh_ref[:, h * _DAUG + dhead : h * _DAUG + dhead + 1] = jnp.ones(
            (br, 1), jnp.bfloat16)
    e1_ref[...] = jnp.dot(wh, a1_ref[...], preferred_element_type=jnp.float32)
    e2 = jnp.dot(wh, a2_ref[...], preferred_element_type=jnp.float32)
    e2t_ref[...] = e2.T


def _project(xin, wcat, a1, a2, br, nheads, dhead):
    npad, din = xin.shape
    dout = wcat.shape[1]
    daug = nheads * _DAUG
    return pl.pallas_call(
        functools.partial(_proj_kernel, nheads=nheads, dhead=dhead),
        grid=(npad // br,),
        in_specs=[
            pl.BlockSpec((br, din), lambda i: (i, 0)),
            pl.BlockSpec((din, dout), lambda i: (0, 0)),
            pl.BlockSpec((dout, _NH), lambda i: (0, 0)),
            pl.BlockSpec((dout, _NH), lambda i: (0, 0)),
        ],
        out_specs=[
            pl.BlockSpec((br, daug), lambda i: (i, 0)),
            pl.BlockSpec((br, _NH), lambda i: (i, 0)),
            pl.BlockSpec((_NH, br), lambda i: (0, i)),
        ],
        out_shape=[
            jax.ShapeDtypeStruct((npad, daug), jnp.bfloat16),
            jax.ShapeDtypeStruct((npad, _NH), jnp.float32),
            jax.ShapeDtypeStruct((_NH, npad), jnp.float32),
        ],
    )(xin, wcat, a1, a2)


def _att_kernel(adj_ref, wh_ref, rs_ref, cs_ref, out_ref,
                *, nheads, dhead, ncols, bc, br, nreal, concat):
    i = pl.program_id(0)
    j = pl.program_id(1)

    @pl.when(j == 0)
    def _init():
        out_ref[...] = jnp.zeros_like(out_ref)

    adj = adj_ref[...]
    adj = jax.lax.cond(
        j == ncols - 1,
        lambda a: jnp.where(
            jax.lax.broadcasted_iota(jnp.int32, a.shape, 1) + j * bc < nreal,
            a, 0.0),
        lambda a: a,
        adj,
    )
    adjb = adj if adj.dtype == jnp.bfloat16 else adj.astype(jnp.bfloat16)

    for h in range(nheads):
        lo = h * _DAUG
        av = rs_ref[:, h : h + 1]
        cv = rs_ref[:, _NH + h : _NH + h + 1]
        bv = cs_ref[h : h + 1, :]
        dv = cs_ref[_NH + h : _NH + h + 1, :]
        p = adjb * jnp.maximum(av * bv, cv * dv)
        out_ref[:, lo : lo + _DAUG] += jnp.dot(
            p, wh_ref[:, lo : lo + _DAUG],
            preferred_element_type=jnp.float32,
        )

    @pl.when(j == ncols - 1)
    def _finish():
        rmask = jax.lax.broadcasted_iota(jnp.int32, (br, 1), 0) + i * br < nreal
        for h in range(nheads):
            lo = h * _DAUG
            den = out_ref[:, lo + dhead : lo + dhead + 1]
            v = out_ref[:, lo : lo + dhead] / (den + _EPS)
            if concat:
                v = jnp.where(v > 0, v, jnp.exp(v) - 1.0)
            out_ref[:, lo : lo + dhead] = jnp.where(rmask, v, 0.0)
            out_ref[:, lo + dhead : lo + _DAUG] = jnp.zeros(
                (br, _DAUG - dhead), jnp.float32)


def _attention(adj, wh, rs, cs, nheads, dhead, concat, br, bc):
    nreal = adj.shape[0]
    npad = wh.shape[0]
    daug = wh.shape[1]
    nrows = npad // br
    ncols = npad // bc
    return pl.pallas_call(
        functools.partial(
            _att_kernel, nheads=nheads, dhead=dhead, ncols=ncols, bc=bc,
            br=br, nreal=nreal, concat=concat,
        ),
        grid=(nrows, ncols),
        in_specs=[
            pl.BlockSpec((br, bc), lambda i, j: (i, j)),
            pl.BlockSpec((bc, daug), lambda i, j: (j, 0)),
            pl.BlockSpec((br, 2 * _NH), lambda i, j: (i, 0)),
            pl.BlockSpec((2 * _NH, bc), lambda i, j: (0, j)),
        ],
        out_specs=pl.BlockSpec((br, daug), lambda i, j: (i, 0)),
        out_shape=jax.ShapeDtypeStruct((npad, daug), jnp.float32),
        compiler_params=pltpu.CompilerParams(
            dimension_semantics=("parallel", "arbitrary")
        ),
    )(adj, wh, rs, cs)


def _leaky(v):
    return jnp.maximum(v, _ALPHA * v)


def _factors(e1, e2t):
    m2 = jnp.max(e2t, axis=1)
    sh = _leaky(e1 + m2[None, :])
    rs = jnp.concatenate(
        [jnp.exp(e1 + m2[None, :] - sh), jnp.exp(_ALPHA * (e1 + m2[None, :]) - sh)],
        axis=1,
    ).astype(jnp.bfloat16)
    cs = jnp.concatenate(
        [jnp.exp(e2t - m2[:, None]), jnp.exp(_ALPHA * (e2t - m2[:, None]))],
        axis=0,
    ).astype(jnp.bfloat16)
    return rs, cs


def kernel(x, adj, W_heads, a_heads, W_out, a_out):
    n, nfeat = x.shape
    nheads, _, nhid = W_heads.shape
    nclass = W_out.shape[1]

    br, bc = 1024, 2048
    npad = -(-n // max(br, bc)) * max(br, bc)

    xp = jnp.pad(x, ((0, npad - n), (0, 0)))

    wcat = jnp.transpose(W_heads, (1, 0, 2)).reshape(nfeat, nheads * nhid)
    a1 = jnp.zeros((nheads * nhid, _NH), jnp.float32)
    a2 = jnp.zeros((nheads * nhid, _NH), jnp.float32)
    for h in range(nheads):
        a1 = a1.at[h * nhid : (h + 1) * nhid, h].set(a_heads[h, :nhid, 0])
        a2 = a2.at[h * nhid : (h + 1) * nhid, h].set(a_heads[h, nhid:, 0])
    wh1, e11, e2t1 = _project(xp, wcat, a1, a2, br, nheads, nhid)
    rs1, cs1 = _factors(e11, e2t1)
    h1 = _attention(adj, wh1, rs1, cs1, nheads, nhid, True, br, bc)

    wcat2 = jnp.zeros((nheads * _DAUG, nclass), jnp.float32)
    for h in range(nheads):
        wcat2 = wcat2.at[h * _DAUG : h * _DAUG + nhid, :].set(
            W_out[h * nhid : (h + 1) * nhid, :])
    b1 = jnp.zeros((nclass, _NH), jnp.float32).at[:, 0].set(a_out[:nclass, 0])
    b2 = jnp.zeros((nclass, _NH), jnp.float32).at[:, 0].set(a_out[nclass:, 0])
    wh2, e12, e2t2 = _project(h1, wcat2, b1, b2, br, 1, nclass)
    rs2, cs2 = _factors(e12, e2t2)
    z = _attention(adj, wh2, rs2, cs2, 1, nclass, True, br, bc)
    return z[:n, :nclass][None, :, :]
